# baseline (speedup 1.0000x reference)
import jax
import jax.numpy as jnp
from jax import lax
from jax.experimental import pallas as pl
from jax.experimental.pallas import tpu as pltpu
from jax.experimental.pallas import tpu_sc as plsc

S = 2048
E = 8
CAP = S // E
D = 1024
DFF = 2048

NC = 2
NS = 16
LANES = 16

TOK_PER_SUB = S // NS
HALF = TOK_PER_SUB // NC
NV = TOK_PER_SUB // LANES

_MESH = plsc.VectorSubcoreMesh(
    core_axis_name="c", subcore_axis_name="s", num_cores=NC, num_subcores=NS
)


def _route_body(eidx_hbm, slot_hbm, ids_v, cnt_v, allcnt_v, base_v,
                slotbuf_v, counts_sh):
  c = lax.axis_index("c")
  s = lax.axis_index("s")
  base128 = s * TOK_PER_SUB
  row0 = pl.multiple_of(base128 + c * HALF, HALF)

  pltpu.sync_copy(eidx_hbm.at[pl.ds(base128, TOK_PER_SUB)], ids_v)
  lane = lax.iota(jnp.int32, LANES)

  cnt = jnp.zeros((LANES,), jnp.int32)
  for v in range(NV):
    ids = ids_v[pl.ds(v * LANES, LANES)]
    for e in range(E):
      p = plsc.all_reduce_population_count(ids == e)
      cnt = jnp.where(lane == e, cnt + p, cnt)
  cnt_v[...] = cnt
  soff = pl.multiple_of(s * LANES, LANES)
  pltpu.sync_copy(cnt_v, counts_sh.at[pl.ds(soff, LANES)])
  plsc.subcore_barrier()
  pltpu.sync_copy(counts_sh, allcnt_v)

  base = jnp.zeros((LANES,), jnp.int32)
  for t in range(NS):
    ct = allcnt_v[pl.ds(t * LANES, LANES)]
    base = base + jnp.where(jnp.full((LANES,), t, jnp.int32) < s, ct, 0)
  base_v[...] = base

  run = jnp.zeros((LANES,), jnp.int32)
  for v in range(NV):
    ids = ids_v[pl.ds(v * LANES, LANES)]
    cnt_v[...] = run
    base_tok = plsc.load_gather(base_v, [ids])
    run_tok = plsc.load_gather(cnt_v, [ids])
    rank = jnp.zeros((LANES,), jnp.int32)
    for e in range(E):
      m = ids == e
      cs = plsc.cumsum(m.astype(jnp.int32))
      rank = jnp.where(m, cs - 1, rank)
      p = plsc.all_reduce_population_count(m)
      run = jnp.where(lane == e, run + p, run)
    pos = base_tok + run_tok + rank
    valid = pos < CAP
    slot = ids * CAP + pos
    slotbuf_v[pl.ds(v * LANES, LANES)] = jnp.where(valid, slot, S)

  off = pl.multiple_of(c * HALF, HALF)
  pltpu.sync_copy(slotbuf_v.at[pl.ds(off, HALF)], slot_hbm.at[pl.ds(row0, HALF)])


_route = pl.kernel(
    _route_body,
    out_type=jax.ShapeDtypeStruct((S,), jnp.int32),
    mesh=_MESH,
    scratch_types=(
        pltpu.VMEM((TOK_PER_SUB,), jnp.int32),
        pltpu.VMEM((LANES,), jnp.int32),
        pltpu.VMEM((NS * LANES,), jnp.int32),
        pltpu.VMEM((LANES,), jnp.int32),
        pltpu.VMEM((TOK_PER_SUB,), jnp.int32),
        pltpu.VMEM_SHARED((NS * LANES,), jnp.int32),
    ),
    compiler_params=pltpu.CompilerParams(needs_layout_passes=False),
)


def _moe_body(ts_ref, hid_ref, w1_ref, b1_ref, w2_ref, b2_ref, o_ref):
  e = pl.program_id(0)
  ts = ts_ref[0]
  col = lax.broadcasted_iota(jnp.int32, (S, CAP), 1) + e * CAP
  m = (ts[:, None] == col).astype(jnp.float32)
  x = lax.dot_general(m, hid_ref[...], (((0,), (0,)), ((), ())),
                      preferred_element_type=jnp.float32)
  h = jnp.maximum(
      jnp.dot(x, w1_ref[0], preferred_element_type=jnp.float32) + b1_ref[0],
      0.0)
  y = jnp.dot(h, w2_ref[0], preferred_element_type=jnp.float32) + b2_ref[0]
  contrib = jnp.dot(m, y, preferred_element_type=jnp.float32)

  @pl.when(e == 0)
  def _():
    o_ref[...] = contrib

  @pl.when(e > 0)
  def _():
    o_ref[...] += contrib


def _moe_tc(ts, hid, w1, b1, w2, b2):
  return pl.pallas_call(
      _moe_body,
      grid=(E,),
      in_specs=[
          pl.BlockSpec((1, S), lambda e: (0, 0)),
          pl.BlockSpec((S, D), lambda e: (0, 0)),
          pl.BlockSpec((1, D, DFF), lambda e: (e, 0, 0)),
          pl.BlockSpec((1, 1, DFF), lambda e: (e, 0, 0)),
          pl.BlockSpec((1, DFF, D), lambda e: (e, 0, 0)),
          pl.BlockSpec((1, 1, D), lambda e: (e, 0, 0)),
      ],
      out_specs=pl.BlockSpec((S, D), lambda e: (0, 0)),
      out_shape=jax.ShapeDtypeStruct((S, D), jnp.float32),
  )(ts, hid, w1, b1.reshape(E, 1, DFF), w2, b2.reshape(E, 1, D))


def kernel(hidden_states, expert_idx, w1, b1, w2, b2):
  hid = hidden_states.reshape(S, D)
  eidx = expert_idx.reshape(S).astype(jnp.int32)
  tok_slot = _route(eidx)
  out = _moe_tc(tok_slot.reshape(1, S), hid, w1, b1, w2, b2)
  return out.reshape(hidden_states.shape)
